# Initial kernel scaffold; baseline (speedup 1.0000x reference)
import jax
import jax.numpy as jnp
from jax.experimental import pallas as pl


def kernel(indices, table):
    raise NotImplementedError("write your pallas kernel here")


# speedup vs baseline: 1.8435x; 1.8435x over previous
import jax
import jax.numpy as jnp
from jax import lax
from jax.experimental import pallas as pl
from jax.experimental.pallas import tpu as pltpu
from jax.experimental.pallas import tpu_sc as plsc

_NUM_EMB = 1000000
_DIM = 64
_B = 16384
_L = 50

_INFO = plsc.get_sparse_core_info()
_NC = _INFO.num_cores
_NS = _INFO.num_subcores
_NW = _NC * _NS

_N = _B * _L
_IW = 128
_NROWS = _N // _IW
_ROWS_PW = _NROWS // _NW
_K = 8
_STEPS = _ROWS_PW // _K
_CHUNK = _K * _IW


def _body(idx_hbm, table_hbm, out_hbm, idx_v, rows_v, sem):
    wid = lax.axis_index("s") * _NC + lax.axis_index("c")
    row0 = wid * _ROWS_PW

    def step(g, carry):
        r0 = row0 + g * _K
        pltpu.sync_copy(idx_hbm.at[pl.ds(r0, _K)], idx_v)
        descs = [
            pltpu.async_copy(
                table_hbm.at[idx_v.at[j]],
                rows_v.at[pl.ds(j * _IW, _IW)],
                sem,
            )
            for j in range(_K)
        ]
        for d in descs:
            d.wait()
        pltpu.sync_copy(rows_v, out_hbm.at[pl.ds(r0 * _IW, _CHUNK)])
        return carry

    lax.fori_loop(0, _STEPS, step, 0)


_mesh = plsc.VectorSubcoreMesh(core_axis_name="c", subcore_axis_name="s")

_gather = pl.kernel(
    _body,
    out_type=jax.ShapeDtypeStruct((_N, _DIM), jnp.float32),
    mesh=_mesh,
    scratch_types=[
        pltpu.VMEM((_K, _IW), jnp.int32),
        pltpu.VMEM((_CHUNK, _DIM), jnp.float32),
        pltpu.SemaphoreType.DMA,
    ],
    compiler_params=pltpu.CompilerParams(use_tc_tiling_on_sc=False),
)


@jax.jit
def kernel(indices, table):
    idx2d = indices.reshape(_NROWS, _IW)
    out = _gather(idx2d, table)
    return out.reshape(_B, _L, _DIM)


# speedup vs baseline: 1.8766x; 1.0180x over previous
import jax
import jax.numpy as jnp
from jax import lax
from jax.experimental import pallas as pl
from jax.experimental.pallas import tpu as pltpu
from jax.experimental.pallas import tpu_sc as plsc

_NUM_EMB = 1000000
_DIM = 64
_B = 16384
_L = 50

_INFO = plsc.get_sparse_core_info()
_NC = _INFO.num_cores
_NS = _INFO.num_subcores
_NW = _NC * _NS

_N = _B * _L
_IW = 128
_NROWS = _N // _IW
_ROWS_PW = _NROWS // _NW
_K = 5
_STEPS = _ROWS_PW // _K
_CHUNK = _K * _IW


def _body(idx_hbm, table_hbm, out_hbm,
          idx0, idx1, rows0, rows1,
          sg0, sg1, sw0, sw1, si0, si1):
    idx_bufs = (idx0, idx1)
    rows_bufs = (rows0, rows1)
    sg = (sg0, sg1)
    sw = (sw0, sw1)
    si = (si0, si1)

    wid = lax.axis_index("s") * _NC + lax.axis_index("c")
    row0 = wid * _ROWS_PW

    pltpu.async_copy(idx_hbm.at[pl.ds(row0, _K)], idx0, si0)
    pltpu.async_copy(idx_hbm.at[pl.ds(row0 + _K, _K)], idx1, si1)

    def two_chunks(h, carry):
        for b in range(2):
            g = h * 2 + b
            r0 = row0 + g * _K
            pltpu.make_async_copy(
                idx_hbm.at[pl.ds(row0, _K)], idx_bufs[b], si[b]).wait()

            @pl.when(g >= 2)
            def _():
                pltpu.make_async_copy(
                    rows_bufs[b], out_hbm.at[pl.ds(r0 * _IW, _CHUNK)],
                    sw[b]).wait()

            for j in range(_K):
                pltpu.async_copy(
                    table_hbm.at[idx_bufs[b].at[j]],
                    rows_bufs[b].at[pl.ds(j * _IW, _IW)],
                    sg[b],
                )
            for j in range(_K):
                pltpu.make_async_copy(
                    table_hbm.at[idx_bufs[b].at[j]],
                    rows_bufs[b].at[pl.ds(j * _IW, _IW)],
                    sg[b],
                ).wait()

            @pl.when(g + 2 < _STEPS)
            def _():
                pltpu.async_copy(
                    idx_hbm.at[pl.ds(r0 + 2 * _K, _K)], idx_bufs[b], si[b])

            pltpu.async_copy(
                rows_bufs[b], out_hbm.at[pl.ds(r0 * _IW, _CHUNK)], sw[b])
        return carry

    lax.fori_loop(0, _STEPS // 2, two_chunks, 0)

    for b in range(2):
        pltpu.make_async_copy(
            rows_bufs[b], out_hbm.at[pl.ds(row0 * _IW, _CHUNK)], sw[b]).wait()


_mesh = plsc.VectorSubcoreMesh(core_axis_name="c", subcore_axis_name="s")

_gather = pl.kernel(
    _body,
    out_type=jax.ShapeDtypeStruct((_N, _DIM), jnp.float32),
    mesh=_mesh,
    scratch_types=[
        pltpu.VMEM((_K, _IW), jnp.int32),
        pltpu.VMEM((_K, _IW), jnp.int32),
        pltpu.VMEM((_CHUNK, _DIM), jnp.float32),
        pltpu.VMEM((_CHUNK, _DIM), jnp.float32),
        pltpu.SemaphoreType.DMA,
        pltpu.SemaphoreType.DMA,
        pltpu.SemaphoreType.DMA,
        pltpu.SemaphoreType.DMA,
        pltpu.SemaphoreType.DMA,
        pltpu.SemaphoreType.DMA,
    ],
    compiler_params=pltpu.CompilerParams(use_tc_tiling_on_sc=False),
)


@jax.jit
def kernel(indices, table):
    idx2d = indices.reshape(_NROWS, _IW)
    out = _gather(idx2d, table)
    return out.reshape(_B, _L, _DIM)
